# Initial kernel scaffold; baseline (speedup 1.0000x reference)
import jax
import jax.numpy as jnp
from jax.experimental import pallas as pl


def kernel(x, edge_index, edge_attr, node_w1, node_b1, node_w2, node_b2, edge_w1, edge_b1, edge_w2, edge_b2, proj_w, proj_b, read_w1, read_b1, read_w2, read_b2, gate_scale, g_gate_center):
    raise NotImplementedError("write your pallas kernel here")


# speedup vs baseline: 5.0016x; 5.0016x over previous
import functools

import jax
import jax.numpy as jnp
from jax import lax
from jax.experimental import pallas as pl
from jax.experimental.pallas import tpu as pltpu
from jax.experimental.pallas import tpu_sc as plsc


def _prep_body(x_ref, nw1_ref, nb1_ref, nw2_ref, nb2_ref, w1ab_ref,
               c_ref, hxsum_ref):
    xb = x_ref[...]
    c_ref[...] = jnp.dot(xb, w1ab_ref[...], preferred_element_type=jnp.float32)
    t = jnp.maximum(
        jnp.dot(xb, nw1_ref[...], preferred_element_type=jnp.float32)
        + nb1_ref[...], 0.0)
    hx = jnp.maximum(
        jnp.dot(t, nw2_ref[...], preferred_element_type=jnp.float32)
        + nb2_ref[...], 0.0)

    @pl.when(pl.program_id(0) == 0)
    def _():
        hxsum_ref[...] = jnp.zeros_like(hxsum_ref)

    hxsum_ref[...] += jnp.sum(hx, axis=0, keepdims=True)


def _edge_body(gsc_ref, gc_ref, ga_ref, gb_ref, ea_ref, w1c_ref, b1_ref,
               w2_ref, b2_ref, projw_ref, projb_ref, hxsum_ref, rw1_ref,
               rb1_ref, rw2_ref, rb2_ref, msum_ref, ec_ref, out_ref, *,
               n_nodes, n_edges, h_dim):
    i = pl.program_id(0)

    @pl.when(i == 0)
    def _():
        msum_ref[...] = jnp.zeros_like(msum_ref)
        ec_ref[...] = jnp.zeros_like(ec_ref)

    ea = ea_ref[...]
    z = (ea[:, 2:3] - gc_ref[...]) * gsc_ref[...]
    gate = jnp.maximum(z, 0.0) + jnp.log(1.0 + jnp.exp(-jnp.abs(z))) + 0.001
    c = jnp.dot(ea, w1c_ref[...], preferred_element_type=jnp.float32)
    h1 = jnp.maximum(
        ga_ref[:, :h_dim] + gb_ref[:, h_dim:] + c + b1_ref[...], 0.0)
    e = jnp.maximum(
        jnp.dot(h1, w2_ref[...], preferred_element_type=jnp.float32)
        + b2_ref[...], 0.0)
    msum_ref[...] += jnp.sum(gate * e, axis=0, keepdims=True)
    p = jnp.dot(ea, projw_ref[...], preferred_element_type=jnp.float32) \
        + projb_ref[...]
    ec_ref[...] += jnp.sum(gate * p, axis=0, keepdims=True)

    @pl.when(i == pl.num_programs(0) - 1)
    def _():
        node_ctx = hxsum_ref[...] * (1.0 / n_nodes) \
            + msum_ref[...] * (2.0 / n_nodes)
        edge_ctx = ec_ref[...] * (1.0 / (n_edges + 1e-6))
        h = jnp.concatenate([node_ctx, edge_ctx], axis=1)
        r1 = jnp.maximum(
            jnp.dot(h, rw1_ref[...], preferred_element_type=jnp.float32)
            + rb1_ref[...], 0.0)
        out_ref[...] = jnp.dot(
            r1, rw2_ref[...], preferred_element_type=jnp.float32) + rb2_ref[...]


_NC, _NS, _L = 2, 16, 16
_NW = _NC * _NS


def _make_sc_gather(n_nodes, w_dim, rows, gpw, r):
    mp = rows * 128
    n_groups = rows // r
    mesh = plsc.VectorSubcoreMesh(core_axis_name="c", subcore_axis_name="s")

    @functools.partial(
        pl.kernel,
        out_type=[
            jax.ShapeDtypeStruct((mp, w_dim), jnp.float32),
            jax.ShapeDtypeStruct((mp, w_dim), jnp.float32),
        ],
        mesh=mesh,
        scratch_types=[
            pltpu.VMEM((r, 128), jnp.int32),
            pltpu.VMEM((r, 128), jnp.int32),
            pltpu.VMEM((r * 128, w_dim), jnp.float32),
            pltpu.VMEM((r * 128, w_dim), jnp.float32),
            pltpu.SemaphoreType.DMA,
            pltpu.SemaphoreType.DMA,
        ],
    )
    def gather_kernel(c_hbm, src_hbm, dst_hbm, ga_hbm, gb_hbm,
                      idxa, idxb, bufa, bufb, sema, semb):
        wid = lax.axis_index("s") * _NC + lax.axis_index("c")

        def body(t, carry):
            g = t * _NW + wid

            @pl.when(g < n_groups)
            def _():
                row = g * r
                base = row * 128
                pltpu.sync_copy(src_hbm.at[pl.ds(row, r)], idxa)
                pltpu.sync_copy(dst_hbm.at[pl.ds(row, r)], idxb)
                cpa = [pltpu.async_copy(c_hbm.at[idxa.at[j]],
                                        bufa.at[pl.ds(j * 128, 128)], sema)
                       for j in range(r)]
                cpb = [pltpu.async_copy(c_hbm.at[idxb.at[j]],
                                        bufb.at[pl.ds(j * 128, 128)], semb)
                       for j in range(r)]
                for cp in cpa:
                    cp.wait()
                pltpu.sync_copy(bufa, ga_hbm.at[pl.ds(base, r * 128)])
                for cp in cpb:
                    cp.wait()
                pltpu.sync_copy(bufb, gb_hbm.at[pl.ds(base, r * 128)])

            return carry

        lax.fori_loop(0, gpw, body, 0)

    return gather_kernel


def kernel(x, edge_index, edge_attr, node_w1, node_b1, node_w2, node_b2,
           edge_w1, edge_b1, edge_w2, edge_b2, proj_w, proj_b, read_w1,
           read_b1, read_w2, read_b2, gate_scale, g_gate_center):
    n, fx = x.shape
    m, fe = edge_attr.shape
    h = node_w1.shape[1]

    w1ab = jnp.concatenate([edge_w1[:fx], edge_w1[fx:2 * fx]], axis=1)
    w1c = edge_w1[2 * fx:]
    wdim = 2 * h

    bn = 2000
    assert n % bn == 0
    prep = pl.pallas_call(
        _prep_body,
        grid=(n // bn,),
        in_specs=[
            pl.BlockSpec((bn, fx), lambda i: (i, 0)),
            pl.BlockSpec((fx, h), lambda i: (0, 0)),
            pl.BlockSpec((1, h), lambda i: (0, 0)),
            pl.BlockSpec((h, h), lambda i: (0, 0)),
            pl.BlockSpec((1, h), lambda i: (0, 0)),
            pl.BlockSpec((fx, wdim), lambda i: (0, 0)),
        ],
        out_specs=[
            pl.BlockSpec((bn, wdim), lambda i: (i, 0)),
            pl.BlockSpec((1, h), lambda i: (0, 0)),
        ],
        out_shape=[
            jax.ShapeDtypeStruct((n, wdim), jnp.float32),
            jax.ShapeDtypeStruct((1, h), jnp.float32),
        ],
    )
    c_tab, hxsum = prep(x, node_w1, node_b1.reshape(1, h), node_w2,
                        node_b2.reshape(1, h), w1ab)

    r = 2
    assert m % (128 * r) == 0
    rows = m // 128
    n_groups = rows // r
    gpw = -(-n_groups // _NW)
    src2d = edge_index[0].reshape(rows, 128)
    dst2d = edge_index[1].reshape(rows, 128)
    ga, gb = _make_sc_gather(n, wdim, rows, gpw, r)(c_tab, src2d, dst2d)

    bm = 2000
    assert m % bm == 0
    egrid = m // bm
    rw2p = jnp.zeros((h, 128), jnp.float32).at[:, :read_w2.shape[1]].set(read_w2)
    rb2p = jnp.zeros((1, 128), jnp.float32).at[:, :read_b2.shape[0]].set(
        read_b2.reshape(1, -1))
    edge_call = pl.pallas_call(
        functools.partial(_edge_body, n_nodes=float(n), n_edges=float(m),
                          h_dim=h),
        grid=(egrid,),
        in_specs=[
            pl.BlockSpec((1, 1), lambda i: (0, 0)),
            pl.BlockSpec((1, 1), lambda i: (0, 0)),
            pl.BlockSpec((bm, wdim), lambda i: (i, 0)),
            pl.BlockSpec((bm, wdim), lambda i: (i, 0)),
            pl.BlockSpec((bm, fe), lambda i: (i, 0)),
            pl.BlockSpec((fe, h), lambda i: (0, 0)),
            pl.BlockSpec((1, h), lambda i: (0, 0)),
            pl.BlockSpec((h, h), lambda i: (0, 0)),
            pl.BlockSpec((1, h), lambda i: (0, 0)),
            pl.BlockSpec((fe, h), lambda i: (0, 0)),
            pl.BlockSpec((1, h), lambda i: (0, 0)),
            pl.BlockSpec((1, h), lambda i: (0, 0)),
            pl.BlockSpec((2 * h, 64), lambda i: (0, 0)),
            pl.BlockSpec((1, 64), lambda i: (0, 0)),
            pl.BlockSpec((h, 128), lambda i: (0, 0)),
            pl.BlockSpec((1, 128), lambda i: (0, 0)),
        ],
        out_specs=[
            pl.BlockSpec((1, h), lambda i: (0, 0)),
            pl.BlockSpec((1, h), lambda i: (0, 0)),
            pl.BlockSpec((1, 128), lambda i: (0, 0)),
        ],
        out_shape=[
            jax.ShapeDtypeStruct((1, h), jnp.float32),
            jax.ShapeDtypeStruct((1, h), jnp.float32),
            jax.ShapeDtypeStruct((1, 128), jnp.float32),
        ],
    )
    gsc = gate_scale.astype(jnp.float32).reshape(1, 1)
    gc = g_gate_center.astype(jnp.float32).reshape(1, 1)
    _, _, out128 = edge_call(
        gsc, gc, ga, gb, edge_attr, w1c, edge_b1.reshape(1, h),
        edge_w2, edge_b2.reshape(1, h), proj_w, proj_b.reshape(1, h), hxsum,
        read_w1, read_b1.reshape(1, 64), rw2p, rb2p)
    return out128[0, :read_w2.shape[1]]


# speedup vs baseline: 5.1357x; 1.0268x over previous
import functools

import jax
import jax.numpy as jnp
from jax import lax
from jax.experimental import pallas as pl
from jax.experimental.pallas import tpu as pltpu
from jax.experimental.pallas import tpu_sc as plsc


def _prep_body(x_ref, nw1_ref, nb1_ref, nw2_ref, nb2_ref, w1ab_ref,
               c_ref, hxsum_ref):
    xb = x_ref[...]
    c_ref[...] = jnp.dot(
        xb, w1ab_ref[...], preferred_element_type=jnp.float32)
    t = jnp.maximum(
        jnp.dot(xb, nw1_ref[...], preferred_element_type=jnp.float32)
        + nb1_ref[...], 0.0)
    hx = jnp.maximum(
        jnp.dot(t, nw2_ref[...], preferred_element_type=jnp.float32)
        + nb2_ref[...], 0.0)

    @pl.when(pl.program_id(0) == 0)
    def _():
        hxsum_ref[...] = jnp.zeros_like(hxsum_ref)

    hxsum_ref[...] += jnp.sum(hx, axis=0, keepdims=True)


def _edge_body(gsc_ref, gc_ref, ga_ref, gb_ref, ea_ref, w1c_ref, b1_ref,
               w2_ref, b2_ref, projw_ref, projb_ref, hxsum_ref, rw1_ref,
               rb1_ref, rw2_ref, rb2_ref, msum_ref, ec_ref, out_ref, *,
               n_nodes, n_edges, h_dim):
    i = pl.program_id(0)

    @pl.when(i == 0)
    def _():
        msum_ref[...] = jnp.zeros_like(msum_ref)
        ec_ref[...] = jnp.zeros_like(ec_ref)

    ea = ea_ref[...]
    z = (ea[:, 2:3] - gc_ref[...]) * gsc_ref[...]
    gate = jnp.maximum(z, 0.0) + jnp.log(1.0 + jnp.exp(-jnp.abs(z))) + 0.001
    c = jnp.dot(ea, w1c_ref[...], preferred_element_type=jnp.float32)
    ab = ga_ref[:, :h_dim].astype(jnp.float32) \
        + gb_ref[:, h_dim:].astype(jnp.float32)
    h1 = jnp.maximum(ab + c + b1_ref[...], 0.0)
    e = jnp.maximum(
        jnp.dot(h1, w2_ref[...], preferred_element_type=jnp.float32)
        + b2_ref[...], 0.0)
    msum_ref[...] += jnp.sum(gate * e, axis=0, keepdims=True)
    p = jnp.dot(ea, projw_ref[...], preferred_element_type=jnp.float32) \
        + projb_ref[...]
    ec_ref[...] += jnp.sum(gate * p, axis=0, keepdims=True)

    @pl.when(i == pl.num_programs(0) - 1)
    def _():
        node_ctx = hxsum_ref[...] * (1.0 / n_nodes) \
            + msum_ref[...] * (2.0 / n_nodes)
        edge_ctx = ec_ref[...] * (1.0 / (n_edges + 1e-6))
        h = jnp.concatenate([node_ctx, edge_ctx], axis=1)
        r1 = jnp.maximum(
            jnp.dot(h, rw1_ref[...], preferred_element_type=jnp.float32)
            + rb1_ref[...], 0.0)
        out_ref[...] = jnp.dot(
            r1, rw2_ref[...], preferred_element_type=jnp.float32) + rb2_ref[...]


_NC, _NS, _L = 2, 16, 16
_NW = _NC * _NS


def _make_sc_gather(w_dim, rows):
    mp = rows * 128
    rpw = 2 * (-(-rows // _NW) // 2 + 1)
    half = rpw // 2
    mesh = plsc.VectorSubcoreMesh(core_axis_name="c", subcore_axis_name="s")

    @functools.partial(
        pl.kernel,
        out_type=[
            jax.ShapeDtypeStruct((mp, w_dim), jnp.float32),
            jax.ShapeDtypeStruct((mp, w_dim), jnp.float32),
        ],
        mesh=mesh,
        scratch_types=[
            pltpu.VMEM((rpw + 8, 128), jnp.int32),
            pltpu.VMEM((rpw + 8, 128), jnp.int32),
            pltpu.VMEM((128, w_dim), jnp.float32),
            pltpu.VMEM((128, w_dim), jnp.float32),
            pltpu.VMEM((128, w_dim), jnp.float32),
            pltpu.VMEM((128, w_dim), jnp.float32),
            pltpu.SemaphoreType.DMA,
            pltpu.SemaphoreType.DMA,
            pltpu.SemaphoreType.DMA,
            pltpu.SemaphoreType.DMA,
            pltpu.SemaphoreType.DMA,
            pltpu.SemaphoreType.DMA,
            pltpu.SemaphoreType.DMA,
            pltpu.SemaphoreType.DMA,
        ],
    )
    def gather_kernel(c_hbm, src_hbm, dst_hbm, ga_hbm, gb_hbm,
                      idxa, idxb, ba0, ba1, bb0, bb1,
                      sga0, sga1, sgb0, sgb1, swa0, swa1, swb0, swb1):
        wid = lax.axis_index("s") * _NC + lax.axis_index("c")
        start = (wid * (rows - rpw)) // (_NW - 1)
        astart = start // 8 * 8
        off = start - astart
        pltpu.sync_copy(src_hbm.at[pl.ds(astart, rpw + 8)], idxa)
        pltpu.sync_copy(dst_hbm.at[pl.ds(astart, rpw + 8)], idxb)

        def step(i, carry):
            s0 = 2 * i + off
            s1 = s0 + 1
            e0 = (astart + s0) * 128
            e1 = (astart + s1) * 128

            @pl.when(i > 0)
            def _():
                pltpu.make_async_copy(
                    ba0, ga_hbm.at[pl.ds(e0, 128)], swa0).wait()
                pltpu.make_async_copy(
                    bb0, gb_hbm.at[pl.ds(e0, 128)], swb0).wait()
                pltpu.make_async_copy(
                    ba1, ga_hbm.at[pl.ds(e1, 128)], swa1).wait()
                pltpu.make_async_copy(
                    bb1, gb_hbm.at[pl.ds(e1, 128)], swb1).wait()

            cpa0 = pltpu.async_copy(c_hbm.at[idxa.at[s0]], ba0, sga0)
            cpb0 = pltpu.async_copy(c_hbm.at[idxb.at[s0]], bb0, sgb0)
            cpa1 = pltpu.async_copy(c_hbm.at[idxa.at[s1]], ba1, sga1)
            cpb1 = pltpu.async_copy(c_hbm.at[idxb.at[s1]], bb1, sgb1)
            cpa0.wait()
            pltpu.async_copy(ba0, ga_hbm.at[pl.ds(e0, 128)], swa0)
            cpb0.wait()
            pltpu.async_copy(bb0, gb_hbm.at[pl.ds(e0, 128)], swb0)
            cpa1.wait()
            pltpu.async_copy(ba1, ga_hbm.at[pl.ds(e1, 128)], swa1)
            cpb1.wait()
            pltpu.async_copy(bb1, gb_hbm.at[pl.ds(e1, 128)], swb1)
            return carry

        lax.fori_loop(0, half, step, 0)
        el = (start + rpw - 2) * 128
        pltpu.make_async_copy(ba0, ga_hbm.at[pl.ds(el, 128)], swa0).wait()
        pltpu.make_async_copy(bb0, gb_hbm.at[pl.ds(el, 128)], swb0).wait()
        pltpu.make_async_copy(
            ba1, ga_hbm.at[pl.ds(el + 128, 128)], swa1).wait()
        pltpu.make_async_copy(
            bb1, gb_hbm.at[pl.ds(el + 128, 128)], swb1).wait()

    return gather_kernel


def kernel(x, edge_index, edge_attr, node_w1, node_b1, node_w2, node_b2,
           edge_w1, edge_b1, edge_w2, edge_b2, proj_w, proj_b, read_w1,
           read_b1, read_w2, read_b2, gate_scale, g_gate_center):
    n, fx = x.shape
    m, fe = edge_attr.shape
    h = node_w1.shape[1]

    w1ab = jnp.concatenate([edge_w1[:fx], edge_w1[fx:2 * fx]], axis=1)
    w1c = edge_w1[2 * fx:]
    wdim = 2 * h

    bn = 2000
    assert n % bn == 0
    prep = pl.pallas_call(
        _prep_body,
        grid=(n // bn,),
        in_specs=[
            pl.BlockSpec((bn, fx), lambda i: (i, 0)),
            pl.BlockSpec((fx, h), lambda i: (0, 0)),
            pl.BlockSpec((1, h), lambda i: (0, 0)),
            pl.BlockSpec((h, h), lambda i: (0, 0)),
            pl.BlockSpec((1, h), lambda i: (0, 0)),
            pl.BlockSpec((fx, wdim), lambda i: (0, 0)),
        ],
        out_specs=[
            pl.BlockSpec((bn, wdim), lambda i: (i, 0)),
            pl.BlockSpec((1, h), lambda i: (0, 0)),
        ],
        out_shape=[
            jax.ShapeDtypeStruct((n, wdim), jnp.float32),
            jax.ShapeDtypeStruct((1, h), jnp.float32),
        ],
    )
    c_tab, hxsum = prep(x, node_w1, node_b1.reshape(1, h), node_w2,
                        node_b2.reshape(1, h), w1ab)

    assert m % 128 == 0
    rows = m // 128
    src2d = jnp.pad(edge_index[0].reshape(rows, 128), ((0, 8), (0, 0)))
    dst2d = jnp.pad(edge_index[1].reshape(rows, 128), ((0, 8), (0, 0)))
    ga, gb = _make_sc_gather(wdim, rows)(c_tab, src2d, dst2d)

    bm = 2000
    assert m % bm == 0
    egrid = m // bm
    rw2p = jnp.zeros((h, 128), jnp.float32).at[:, :read_w2.shape[1]].set(read_w2)
    rb2p = jnp.zeros((1, 128), jnp.float32).at[:, :read_b2.shape[0]].set(
        read_b2.reshape(1, -1))
    edge_call = pl.pallas_call(
        functools.partial(_edge_body, n_nodes=float(n), n_edges=float(m),
                          h_dim=h),
        grid=(egrid,),
        in_specs=[
            pl.BlockSpec((1, 1), lambda i: (0, 0)),
            pl.BlockSpec((1, 1), lambda i: (0, 0)),
            pl.BlockSpec((bm, wdim), lambda i: (i, 0)),
            pl.BlockSpec((bm, wdim), lambda i: (i, 0)),
            pl.BlockSpec((bm, fe), lambda i: (i, 0)),
            pl.BlockSpec((fe, h), lambda i: (0, 0)),
            pl.BlockSpec((1, h), lambda i: (0, 0)),
            pl.BlockSpec((h, h), lambda i: (0, 0)),
            pl.BlockSpec((1, h), lambda i: (0, 0)),
            pl.BlockSpec((fe, h), lambda i: (0, 0)),
            pl.BlockSpec((1, h), lambda i: (0, 0)),
            pl.BlockSpec((1, h), lambda i: (0, 0)),
            pl.BlockSpec((2 * h, 64), lambda i: (0, 0)),
            pl.BlockSpec((1, 64), lambda i: (0, 0)),
            pl.BlockSpec((h, 128), lambda i: (0, 0)),
            pl.BlockSpec((1, 128), lambda i: (0, 0)),
        ],
        out_specs=[
            pl.BlockSpec((1, h), lambda i: (0, 0)),
            pl.BlockSpec((1, h), lambda i: (0, 0)),
            pl.BlockSpec((1, 128), lambda i: (0, 0)),
        ],
        out_shape=[
            jax.ShapeDtypeStruct((1, h), jnp.float32),
            jax.ShapeDtypeStruct((1, h), jnp.float32),
            jax.ShapeDtypeStruct((1, 128), jnp.float32),
        ],
    )
    gsc = gate_scale.astype(jnp.float32).reshape(1, 1)
    gc = g_gate_center.astype(jnp.float32).reshape(1, 1)
    _, _, out128 = edge_call(
        gsc, gc, ga, gb, edge_attr, w1c, edge_b1.reshape(1, h),
        edge_w2, edge_b2.reshape(1, h), proj_w, proj_b.reshape(1, h), hxsum,
        read_w1, read_b1.reshape(1, 64), rw2p, rb2p)
    return out128[0, :read_w2.shape[1]]


# speedup vs baseline: 6.6667x; 1.2981x over previous
import functools

import jax
import jax.numpy as jnp
from jax import lax
from jax.experimental import pallas as pl
from jax.experimental.pallas import tpu as pltpu
from jax.experimental.pallas import tpu_sc as plsc


def _prep_body(x_ref, nw1_ref, nb1_ref, nw2_ref, nb2_ref, w1ab_ref,
               c_ref, hxsum_ref):
    xb = x_ref[...]
    cf = jnp.dot(xb, w1ab_ref[...], preferred_element_type=jnp.float32)
    cu = lax.bitcast_convert_type(cf, jnp.uint32)
    cb = (cu + 0x7FFF + ((cu >> 16) & 1)) >> 16
    nh = cf.shape[1] // 2
    c_ref[...] = lax.bitcast_convert_type(
        cb[:, :nh] | (cb[:, nh:] << 16), jnp.int32)
    t = jnp.maximum(
        jnp.dot(xb, nw1_ref[...], preferred_element_type=jnp.float32)
        + nb1_ref[...], 0.0)
    hx = jnp.maximum(
        jnp.dot(t, nw2_ref[...], preferred_element_type=jnp.float32)
        + nb2_ref[...], 0.0)

    @pl.when(pl.program_id(0) == 0)
    def _():
        hxsum_ref[...] = jnp.zeros_like(hxsum_ref)

    hxsum_ref[...] += jnp.sum(hx, axis=0, keepdims=True)


def _edge_body(gsc_ref, gc_ref, ga_ref, gb_ref, ea_ref, w1c_ref, b1_ref,
               w2_ref, b2_ref, projw_ref, projb_ref, hxsum_ref, rw1_ref,
               rb1_ref, rw2_ref, rb2_ref, msum_ref, ec_ref, out_ref, *,
               n_nodes, n_edges, h_dim):
    i = pl.program_id(0)

    @pl.when(i == 0)
    def _():
        msum_ref[...] = jnp.zeros_like(msum_ref)
        ec_ref[...] = jnp.zeros_like(ec_ref)

    ea = ea_ref[...]
    z = (ea[:, 2:3] - gc_ref[...]) * gsc_ref[...]
    gate = jnp.maximum(z, 0.0) + jnp.log(1.0 + jnp.exp(-jnp.abs(z))) + 0.001
    c = jnp.dot(ea, w1c_ref[...], preferred_element_type=jnp.float32)
    gau = lax.bitcast_convert_type(ga_ref[...], jnp.uint32)
    gbu = lax.bitcast_convert_type(gb_ref[...], jnp.uint32)
    a = lax.bitcast_convert_type(gau << 16, jnp.float32)
    b = lax.bitcast_convert_type(gbu & jnp.uint32(0xFFFF0000), jnp.float32)
    h1 = jnp.maximum(a + b + c + b1_ref[...], 0.0)
    e = jnp.maximum(
        jnp.dot(h1, w2_ref[...], preferred_element_type=jnp.float32)
        + b2_ref[...], 0.0)
    msum_ref[...] += jnp.sum(gate * e, axis=0, keepdims=True)
    p = jnp.dot(ea, projw_ref[...], preferred_element_type=jnp.float32) \
        + projb_ref[...]
    ec_ref[...] += jnp.sum(gate * p, axis=0, keepdims=True)

    @pl.when(i == pl.num_programs(0) - 1)
    def _():
        node_ctx = hxsum_ref[...] * (1.0 / n_nodes) \
            + msum_ref[...] * (2.0 / n_nodes)
        edge_ctx = ec_ref[...] * (1.0 / (n_edges + 1e-6))
        h = jnp.concatenate([node_ctx, edge_ctx], axis=1)
        r1 = jnp.maximum(
            jnp.dot(h, rw1_ref[...], preferred_element_type=jnp.float32)
            + rb1_ref[...], 0.0)
        out_ref[...] = jnp.dot(
            r1, rw2_ref[...], preferred_element_type=jnp.float32) + rb2_ref[...]


_NC, _NS, _L = 2, 16, 16
_NW = _NC * _NS


def _make_sc_gather(w_dim, rows, n_nodes):
    mp = rows * 128
    rpw = 2 * (-(-rows // _NW) // 2 + 1)
    half = rpw // 2
    cpw = 8 * (-(-n_nodes // _NS) // 8 + 2)
    cstep = (n_nodes - cpw) // (_NS - 1) // 8 * 8
    assert (_NS - 1) * cstep + cpw >= n_nodes and cstep < cpw
    mesh = plsc.VectorSubcoreMesh(core_axis_name="c", subcore_axis_name="s")

    @functools.partial(
        pl.kernel,
        out_type=[
            jax.ShapeDtypeStruct((mp, w_dim), jnp.int32),
            jax.ShapeDtypeStruct((mp, w_dim), jnp.int32),
        ],
        mesh=mesh,
        scratch_types=[
            pltpu.VMEM_SHARED((n_nodes, w_dim), jnp.int32),
            pltpu.VMEM((rpw + 8, 128), jnp.int32),
            pltpu.VMEM((rpw + 8, 128), jnp.int32),
            pltpu.VMEM((128, w_dim), jnp.int32),
            pltpu.VMEM((128, w_dim), jnp.int32),
            pltpu.VMEM((128, w_dim), jnp.int32),
            pltpu.VMEM((128, w_dim), jnp.int32),
            pltpu.SemaphoreType.DMA,
            pltpu.SemaphoreType.DMA,
            pltpu.SemaphoreType.DMA,
            pltpu.SemaphoreType.DMA,
            pltpu.SemaphoreType.DMA,
            pltpu.SemaphoreType.DMA,
            pltpu.SemaphoreType.DMA,
            pltpu.SemaphoreType.DMA,
        ],
    )
    def gather_kernel(c_hbm, src_hbm, dst_hbm, ga_hbm, gb_hbm,
                      c_sp, idxa, idxb, ba0, ba1, bb0, bb1,
                      sga0, sga1, sgb0, sgb1, swa0, swa1, swb0, swb1):
        sid = lax.axis_index("s")
        wid = sid * _NC + lax.axis_index("c")
        cstart = sid * cstep
        pltpu.sync_copy(c_hbm.at[pl.ds(cstart, cpw)],
                        c_sp.at[pl.ds(cstart, cpw)])
        plsc.subcore_barrier()
        start = (wid * (rows - rpw)) // (_NW - 1)
        astart = start // 8 * 8
        off = start - astart
        pltpu.sync_copy(src_hbm.at[pl.ds(astart, rpw + 8)], idxa)
        pltpu.sync_copy(dst_hbm.at[pl.ds(astart, rpw + 8)], idxb)

        def step(i, carry):
            s0 = 2 * i + off
            s1 = s0 + 1
            e0 = (astart + s0) * 128
            e1 = (astart + s1) * 128

            @pl.when(i > 0)
            def _():
                pltpu.make_async_copy(
                    ba0, ga_hbm.at[pl.ds(e0, 128)], swa0).wait()
                pltpu.make_async_copy(
                    bb0, gb_hbm.at[pl.ds(e0, 128)], swb0).wait()
                pltpu.make_async_copy(
                    ba1, ga_hbm.at[pl.ds(e1, 128)], swa1).wait()
                pltpu.make_async_copy(
                    bb1, gb_hbm.at[pl.ds(e1, 128)], swb1).wait()

            cpa0 = pltpu.async_copy(c_sp.at[idxa.at[s0]], ba0, sga0)
            cpb0 = pltpu.async_copy(c_sp.at[idxb.at[s0]], bb0, sgb0)
            cpa1 = pltpu.async_copy(c_sp.at[idxa.at[s1]], ba1, sga1)
            cpb1 = pltpu.async_copy(c_sp.at[idxb.at[s1]], bb1, sgb1)
            cpa0.wait()
            pltpu.async_copy(ba0, ga_hbm.at[pl.ds(e0, 128)], swa0)
            cpb0.wait()
            pltpu.async_copy(bb0, gb_hbm.at[pl.ds(e0, 128)], swb0)
            cpa1.wait()
            pltpu.async_copy(ba1, ga_hbm.at[pl.ds(e1, 128)], swa1)
            cpb1.wait()
            pltpu.async_copy(bb1, gb_hbm.at[pl.ds(e1, 128)], swb1)
            return carry

        lax.fori_loop(0, half, step, 0)
        el = (start + rpw - 2) * 128
        pltpu.make_async_copy(ba0, ga_hbm.at[pl.ds(el, 128)], swa0).wait()
        pltpu.make_async_copy(bb0, gb_hbm.at[pl.ds(el, 128)], swb0).wait()
        pltpu.make_async_copy(
            ba1, ga_hbm.at[pl.ds(el + 128, 128)], swa1).wait()
        pltpu.make_async_copy(
            bb1, gb_hbm.at[pl.ds(el + 128, 128)], swb1).wait()

    return gather_kernel


def kernel(x, edge_index, edge_attr, node_w1, node_b1, node_w2, node_b2,
           edge_w1, edge_b1, edge_w2, edge_b2, proj_w, proj_b, read_w1,
           read_b1, read_w2, read_b2, gate_scale, g_gate_center):
    n, fx = x.shape
    m, fe = edge_attr.shape
    h = node_w1.shape[1]

    w1ab = jnp.concatenate([edge_w1[:fx], edge_w1[fx:2 * fx]], axis=1)
    w1c = edge_w1[2 * fx:]
    wdim = 2 * h

    bn = 2000
    assert n % bn == 0
    prep = pl.pallas_call(
        _prep_body,
        grid=(n // bn,),
        in_specs=[
            pl.BlockSpec((bn, fx), lambda i: (i, 0)),
            pl.BlockSpec((fx, h), lambda i: (0, 0)),
            pl.BlockSpec((1, h), lambda i: (0, 0)),
            pl.BlockSpec((h, h), lambda i: (0, 0)),
            pl.BlockSpec((1, h), lambda i: (0, 0)),
            pl.BlockSpec((fx, wdim), lambda i: (0, 0)),
        ],
        out_specs=[
            pl.BlockSpec((bn, h), lambda i: (i, 0)),
            pl.BlockSpec((1, h), lambda i: (0, 0)),
        ],
        out_shape=[
            jax.ShapeDtypeStruct((n, h), jnp.int32),
            jax.ShapeDtypeStruct((1, h), jnp.float32),
        ],
    )
    c_tab, hxsum = prep(x, node_w1, node_b1.reshape(1, h), node_w2,
                        node_b2.reshape(1, h), w1ab)

    assert m % 128 == 0
    rows = m // 128
    src2d = jnp.pad(edge_index[0].reshape(rows, 128), ((0, 8), (0, 0)))
    dst2d = jnp.pad(edge_index[1].reshape(rows, 128), ((0, 8), (0, 0)))
    ga, gb = _make_sc_gather(h, rows, n)(c_tab, src2d, dst2d)

    bm = 2000
    assert m % bm == 0
    egrid = m // bm
    rw2p = jnp.zeros((h, 128), jnp.float32).at[:, :read_w2.shape[1]].set(read_w2)
    rb2p = jnp.zeros((1, 128), jnp.float32).at[:, :read_b2.shape[0]].set(
        read_b2.reshape(1, -1))
    edge_call = pl.pallas_call(
        functools.partial(_edge_body, n_nodes=float(n), n_edges=float(m),
                          h_dim=h),
        grid=(egrid,),
        in_specs=[
            pl.BlockSpec((1, 1), lambda i: (0, 0)),
            pl.BlockSpec((1, 1), lambda i: (0, 0)),
            pl.BlockSpec((bm, h), lambda i: (i, 0)),
            pl.BlockSpec((bm, h), lambda i: (i, 0)),
            pl.BlockSpec((bm, fe), lambda i: (i, 0)),
            pl.BlockSpec((fe, h), lambda i: (0, 0)),
            pl.BlockSpec((1, h), lambda i: (0, 0)),
            pl.BlockSpec((h, h), lambda i: (0, 0)),
            pl.BlockSpec((1, h), lambda i: (0, 0)),
            pl.BlockSpec((fe, h), lambda i: (0, 0)),
            pl.BlockSpec((1, h), lambda i: (0, 0)),
            pl.BlockSpec((1, h), lambda i: (0, 0)),
            pl.BlockSpec((2 * h, 64), lambda i: (0, 0)),
            pl.BlockSpec((1, 64), lambda i: (0, 0)),
            pl.BlockSpec((h, 128), lambda i: (0, 0)),
            pl.BlockSpec((1, 128), lambda i: (0, 0)),
        ],
        out_specs=[
            pl.BlockSpec((1, h), lambda i: (0, 0)),
            pl.BlockSpec((1, h), lambda i: (0, 0)),
            pl.BlockSpec((1, 128), lambda i: (0, 0)),
        ],
        out_shape=[
            jax.ShapeDtypeStruct((1, h), jnp.float32),
            jax.ShapeDtypeStruct((1, h), jnp.float32),
            jax.ShapeDtypeStruct((1, 128), jnp.float32),
        ],
    )
    gsc = gate_scale.astype(jnp.float32).reshape(1, 1)
    gc = g_gate_center.astype(jnp.float32).reshape(1, 1)
    _, _, out128 = edge_call(
        gsc, gc, ga, gb, edge_attr, w1c, edge_b1.reshape(1, h),
        edge_w2, edge_b2.reshape(1, h), proj_w, proj_b.reshape(1, h), hxsum,
        read_w1, read_b1.reshape(1, 64), rw2p, rb2p)
    return out128[0, :read_w2.shape[1]]
